# baseline (speedup 1.0000x reference)
import functools

import jax
import jax.numpy as jnp
from jax import lax
from jax.experimental import pallas as pl
from jax.experimental.pallas import tpu as pltpu
from jax.experimental.pallas import tpu_sc as plsc

_C0 = 104
_K = 4


@functools.lru_cache(maxsize=None)
def _make_gather(V, D, S, H, HP):
    info = plsc.get_sparse_core_info()
    NC, NS = info.num_cores, info.num_subcores
    NW = NC * NS
    SH = S // 2
    assert SH % NW == 0
    seqs_per_w = SH // NW
    C1 = H - _C0
    sizes = (_C0, C1, _C0, C1)
    offs = (0, _C0, 0, _C0)

    mesh = plsc.VectorSubcoreMesh(core_axis_name="c", subcore_axis_name="s")

    @functools.partial(
        pl.kernel,
        mesh=mesh,
        out_type=jax.ShapeDtypeStruct((SH * H, 2 * D), jnp.float32),
        scratch_types=[
            pltpu.VMEM((2 * seqs_per_w, _C0), jnp.int32),
            pltpu.VMEM((2 * seqs_per_w, C1), jnp.int32),
            pltpu.VMEM((2, _K, _C0, D), jnp.float32),
            pltpu.SemaphoreType.DMA,
            pltpu.SemaphoreType.DMA,
            pltpu.SemaphoreType.DMA,
            pltpu.SemaphoreType.DMA,
            pltpu.SemaphoreType.DMA,
        ],
        compiler_params=pltpu.CompilerParams(use_tc_tiling_on_sc=False),
    )
    def gather(table_hbm, x_hbm, out_hbm, idx_a, idx_b, rows_v, g0, g1, w0, w1, ssem):
        wid = lax.axis_index("s") * NC + lax.axis_index("c")
        s_base = wid * seqs_per_w

        def stage_fire(g, carry):
            for side in range(2):
                s = s_base + g + side * SH
                pltpu.async_copy(
                    x_hbm.at[s, pl.ds(0, _C0)], idx_a.at[side * seqs_per_w + g], ssem
                )
                pltpu.async_copy(
                    x_hbm.at[s, pl.ds(_C0, C1)], idx_b.at[side * seqs_per_w + g], ssem
                )
            return carry

        def stage_drain(g, carry):
            for side in range(2):
                pltpu.make_async_copy(
                    x_hbm.at[0, pl.ds(0, _C0)], idx_a.at[side * seqs_per_w + g], ssem
                ).wait()
                pltpu.make_async_copy(
                    x_hbm.at[0, pl.ds(_C0, C1)], idx_b.at[side * seqs_per_w + g], ssem
                ).wait()
            return carry

        lax.fori_loop(0, seqs_per_w, stage_fire, 0)
        lax.fori_loop(0, seqs_per_w, stage_drain, 0)

        def idx_ref(g, b):
            side = b // 2
            arr = idx_a if b % 2 == 0 else idx_b
            return arr.at[side * seqs_per_w + g]

        def fire_g(g, h, sem):
            for b in range(_K):
                pltpu.async_copy(
                    table_hbm.at[idx_ref(g, b)],
                    rows_v.at[h, b, pl.ds(0, sizes[b])],
                    sem,
                )

        def drain_g(h, sem):
            for b in range(_K):
                pltpu.make_async_copy(
                    table_hbm.at[pl.ds(0, sizes[b])],
                    rows_v.at[h, b, pl.ds(0, sizes[b])],
                    sem,
                ).wait()

        def fire_w(g, h, sem):
            for b in range(_K):
                q0 = (s_base + g) * H + offs[b]
                pltpu.async_copy(
                    rows_v.at[h, b, pl.ds(0, sizes[b])],
                    out_hbm.at[pl.ds(q0, sizes[b]), pl.ds((b // 2) * D, D)],
                    sem,
                )

        def drain_w(h, sem):
            for b in range(_K):
                pltpu.make_async_copy(
                    rows_v.at[h, b, pl.ds(0, sizes[b])],
                    out_hbm.at[pl.ds(0, sizes[b]), pl.ds(0, D)],
                    sem,
                ).wait()

        fire_g(0, 0, g0)

        def body(t, carry):
            @pl.when(t > 0)
            def _():
                drain_w(1, w1)

            fire_g(2 * t + 1, 1, g1)
            drain_g(0, g0)
            fire_w(2 * t, 0, w0)
            drain_w(0, w0)

            @pl.when(t < seqs_per_w // 2 - 1)
            def _():
                fire_g(2 * t + 2, 0, g0)

            drain_g(1, g1)
            fire_w(2 * t + 1, 1, w1)
            return carry

        lax.fori_loop(0, seqs_per_w // 2, body, 0)
        drain_w(1, w1)

    return gather


@functools.lru_cache(maxsize=None)
def _make_relayout(S, H, D):
    SH = S // 2
    n_seq = 64
    n_blk = SH // n_seq

    def body(in_ref, out_ref):
        p = pl.program_id(1)
        y = in_ref[...]

        @pl.when(p == 0)
        def _():
            for t in range(n_seq):
                out_ref[t] = y[t * H : (t + 1) * H, 0:D]

        @pl.when(p == 1)
        def _():
            for t in range(n_seq):
                out_ref[t] = y[t * H : (t + 1) * H, D : 2 * D]

    return pl.pallas_call(
        body,
        grid=(n_blk, 2),
        in_specs=[
            pl.BlockSpec((n_seq * H, 2 * D), lambda s, p: (s, 0)),
        ],
        out_specs=pl.BlockSpec((n_seq, H, D), lambda s, p: (p * n_blk + s, 0, 0)),
        out_shape=jax.ShapeDtypeStruct((S, H, D), jnp.float32),
    )


def kernel(x, W):
    S, H = x.shape
    V, D = W.shape
    HP = (H + 127) // 128 * 128
    xp = jnp.pad(x, ((0, 0), (0, HP - H)))
    pairs = _make_gather(V, D, S, H, HP)(W, xp)
    return _make_relayout(S, H, D)(pairs)


# speedup vs baseline: 1.0181x; 1.0181x over previous
import functools

import jax
import jax.numpy as jnp
from jax import lax
from jax.experimental import pallas as pl
from jax.experimental.pallas import tpu as pltpu
from jax.experimental.pallas import tpu_sc as plsc

_C0 = 104
_K = 4


@functools.lru_cache(maxsize=None)
def _make_gather(V, D, S, H, HP):
    info = plsc.get_sparse_core_info()
    NC, NS = info.num_cores, info.num_subcores
    NW = NC * NS
    assert S % (2 * NW) == 0
    seqs_per_w = S // NW
    C1 = H - _C0
    sizes = (_C0, C1, _C0, C1)
    offs = (0, _C0, 0, _C0)

    mesh = plsc.VectorSubcoreMesh(core_axis_name="c", subcore_axis_name="s")

    @functools.partial(
        pl.kernel,
        mesh=mesh,
        out_type=jax.ShapeDtypeStruct((S, H, D), jnp.float32),
        scratch_types=[
            pltpu.VMEM((seqs_per_w, _C0), jnp.int32),
            pltpu.VMEM((seqs_per_w, C1), jnp.int32),
            pltpu.VMEM((2, _K, _C0, D), jnp.float32),
            pltpu.SemaphoreType.DMA,
            pltpu.SemaphoreType.DMA,
            pltpu.SemaphoreType.DMA,
            pltpu.SemaphoreType.DMA,
            pltpu.SemaphoreType.DMA,
        ],
        compiler_params=pltpu.CompilerParams(use_tc_tiling_on_sc=False),
    )
    def gather(table_hbm, x_hbm, out_hbm, idx_a, idx_b, rows_v, g0, g1, w0, w1, ssem):
        wid = lax.axis_index("s") * NC + lax.axis_index("c")
        s_base = wid * seqs_per_w

        def stage_fire(sl, carry):
            s = s_base + sl
            pltpu.async_copy(x_hbm.at[s, pl.ds(0, _C0)], idx_a.at[sl], ssem)
            pltpu.async_copy(x_hbm.at[s, pl.ds(_C0, C1)], idx_b.at[sl], ssem)
            return carry

        def stage_drain(sl, carry):
            pltpu.make_async_copy(
                x_hbm.at[0, pl.ds(0, _C0)], idx_a.at[sl], ssem
            ).wait()
            pltpu.make_async_copy(
                x_hbm.at[0, pl.ds(_C0, C1)], idx_b.at[sl], ssem
            ).wait()
            return carry

        lax.fori_loop(0, seqs_per_w, stage_fire, 0)
        lax.fori_loop(0, seqs_per_w, stage_drain, 0)

        def idx_ref(g, b):
            arr = idx_a if b % 2 == 0 else idx_b
            return arr.at[2 * g + b // 2]

        def fire_g(g, h, sem):
            for b in range(_K):
                pltpu.async_copy(
                    table_hbm.at[idx_ref(g, b)],
                    rows_v.at[h, b, pl.ds(0, sizes[b])],
                    sem,
                )

        def drain_g(h, sem):
            for b in range(_K):
                pltpu.make_async_copy(
                    table_hbm.at[pl.ds(0, sizes[b])],
                    rows_v.at[h, b, pl.ds(0, sizes[b])],
                    sem,
                ).wait()

        def fire_w(g, h, sem):
            for b in range(_K):
                pltpu.async_copy(
                    rows_v.at[h, b, pl.ds(0, sizes[b])],
                    out_hbm.at[s_base + 2 * g + b // 2, pl.ds(offs[b], sizes[b])],
                    sem,
                )

        def drain_w(h, sem):
            for b in range(_K):
                pltpu.make_async_copy(
                    rows_v.at[h, b, pl.ds(0, sizes[b])],
                    out_hbm.at[0, pl.ds(0, sizes[b])],
                    sem,
                ).wait()

        fire_g(0, 0, g0)

        n_groups = seqs_per_w // 2

        def body(t, carry):
            @pl.when(t > 0)
            def _():
                drain_w(1, w1)

            fire_g(2 * t + 1, 1, g1)
            drain_g(0, g0)
            fire_w(2 * t, 0, w0)
            drain_w(0, w0)

            @pl.when(t < n_groups // 2 - 1)
            def _():
                fire_g(2 * t + 2, 0, g0)

            drain_g(1, g1)
            fire_w(2 * t + 1, 1, w1)
            return carry

        lax.fori_loop(0, n_groups // 2, body, 0)
        drain_w(1, w1)

    return gather


def kernel(x, W):
    S, H = x.shape
    V, D = W.shape
    HP = (H + 127) // 128 * 128
    xp = jnp.pad(x, ((0, 0), (0, HP - H)))
    return _make_gather(V, D, S, H, HP)(W, xp)
